# Initial kernel scaffold; baseline (speedup 1.0000x reference)
import jax
import jax.numpy as jnp
from jax.experimental import pallas as pl


def kernel(x, tables):
    raise NotImplementedError("write your pallas kernel here")


# speedup vs baseline: 1.0962x; 1.0962x over previous
import jax
import jax.numpy as jnp
from jax import lax
from jax.experimental import pallas as pl
from jax.experimental.pallas import tpu as pltpu
from jax.experimental.pallas import tpu_sc as plsc

N_FIELDS_C = 26
VOCAB_C = 100000
D_C = 32
NC, NS, L = 2, 16, 16
NW = NC * NS
CHUNK = 128


def _make_kernel(total: int):
    per_w = total // NW
    nch = per_w // CHUNK

    def body(x_hbm, tab_hbm, out_hbm, gidx_v, rows_v, gsem, wsem):
        wid = lax.axis_index("s") * NC + lax.axis_index("c")
        r0 = wid * nch
        base = wid * per_w

        pltpu.sync_copy(x_hbm.at[pl.ds(r0, nch)], gidx_v)

        lane = lax.iota(jnp.int32, L)

        def ixbody(rr, carry):
            p0 = (r0 + rr) * CHUNK
            for c in range(CHUNK // L):
                pos = p0 + c * L + lane
                f = pos % N_FIELDS_C
                sl = pl.ds(c * L, L)
                gidx_v[rr, sl] = gidx_v[rr, sl] + f * VOCAB_C
            return carry

        lax.fori_loop(0, nch, ixbody, 0)

        def gbody(cc, carry):
            pltpu.async_copy(tab_hbm.at[gidx_v.at[cc]], rows_v, gsem).wait()
            pltpu.async_copy(
                rows_v, out_hbm.at[pl.ds(base + cc * CHUNK, CHUNK)], wsem
            ).wait()
            return carry

        lax.fori_loop(0, nch, gbody, 0)

    return pl.kernel(
        body,
        out_type=jax.ShapeDtypeStruct((total, D_C), jnp.float32),
        mesh=plsc.VectorSubcoreMesh(
            core_axis_name="c", subcore_axis_name="s",
            num_cores=NC, num_subcores=NS,
        ),
        scratch_types=[
            pltpu.VMEM((nch, CHUNK), jnp.int32),
            pltpu.VMEM((CHUNK, D_C), jnp.float32),
            pltpu.SemaphoreType.DMA,
            pltpu.SemaphoreType.DMA,
        ],
        compiler_params=pltpu.CompilerParams(use_tc_tiling_on_sc=False),
    )


def kernel(x, tables):
    b, nf = x.shape
    _, vocab, d = tables.shape
    total = b * nf
    x_flat = x.astype(jnp.int32).reshape(total // CHUNK, CHUNK)
    tab = tables.reshape(nf * vocab, d)
    out = _make_kernel(total)(x_flat, tab)
    return out.reshape(b, nf, d)


# speedup vs baseline: 1.1544x; 1.0531x over previous
import jax
import jax.numpy as jnp
from jax import lax
from jax.experimental import pallas as pl
from jax.experimental.pallas import tpu as pltpu
from jax.experimental.pallas import tpu_sc as plsc

N_FIELDS_C = 26
VOCAB_C = 100000
D_C = 32
NC, NS, L = 2, 16, 16
NW = NC * NS
CHUNK = 128
GSZ = 8
GROWS = GSZ * CHUNK


def _make_kernel(total: int):
    per_w = total // NW
    nch = per_w // CHUNK
    ng = nch // GSZ

    def body(x_hbm, tab_hbm, out_hbm, gidx_v, rows0, rows1,
             gs0, gs1, ws0, ws1):
        wid = lax.axis_index("s") * NC + lax.axis_index("c")
        r0 = wid * nch
        base = wid * per_w

        pltpu.sync_copy(x_hbm.at[pl.ds(r0, nch)], gidx_v)

        lane = lax.iota(jnp.int32, L)

        def compute_group_indices(g):
            for j in range(GSZ):
                ch = g * GSZ + j
                p0 = (r0 + ch) * CHUNK
                for c in range(CHUNK // L):
                    pos = p0 + c * L + lane
                    f = pos % N_FIELDS_C
                    sl = pl.ds(c * L, L)
                    gidx_v[ch, sl] = gidx_v[ch, sl] + f * VOCAB_C

        def fire_group(g, buf, gsem):
            for j in range(GSZ):
                ch = g * GSZ + j
                pltpu.async_copy(tab_hbm.at[gidx_v.at[ch]],
                                 buf.at[pl.ds(j * CHUNK, CHUNK)], gsem)

        def wait_group(buf, gsem):
            pltpu.make_async_copy(tab_hbm.at[pl.ds(0, GROWS)], buf, gsem).wait()

        def fire_write(g, buf, wsem):
            pltpu.async_copy(buf, out_hbm.at[pl.ds(base + g * GROWS, GROWS)],
                             wsem)

        def wait_write(buf, wsem):
            pltpu.make_async_copy(buf, out_hbm.at[pl.ds(base, GROWS)],
                                  wsem).wait()

        compute_group_indices(0)
        fire_group(0, rows0, gs0)

        def gloop(g, carry):
            even = (g % 2) == 0
            nxt = g + 1 < ng

            @pl.when(nxt)
            def _():
                compute_group_indices(g + 1)

            @pl.when(jnp.logical_and(nxt, even))
            def _():
                @pl.when(g >= 1)
                def _():
                    wait_write(rows1, ws1)
                fire_group(g + 1, rows1, gs1)

            @pl.when(jnp.logical_and(nxt, jnp.logical_not(even)))
            def _():
                wait_write(rows0, ws0)
                fire_group(g + 1, rows0, gs0)

            @pl.when(even)
            def _():
                wait_group(rows0, gs0)
                fire_write(g, rows0, ws0)

            @pl.when(jnp.logical_not(even))
            def _():
                wait_group(rows1, gs1)
                fire_write(g, rows1, ws1)

            return carry

        lax.fori_loop(0, ng, gloop, 0)

        wait_write(rows1, ws1)
        wait_write(rows0, ws0)

    return pl.kernel(
        body,
        out_type=jax.ShapeDtypeStruct((total, D_C), jnp.float32),
        mesh=plsc.VectorSubcoreMesh(
            core_axis_name="c", subcore_axis_name="s",
            num_cores=NC, num_subcores=NS,
        ),
        scratch_types=[
            pltpu.VMEM((nch, CHUNK), jnp.int32),
            pltpu.VMEM((GROWS, D_C), jnp.float32),
            pltpu.VMEM((GROWS, D_C), jnp.float32),
            pltpu.SemaphoreType.DMA,
            pltpu.SemaphoreType.DMA,
            pltpu.SemaphoreType.DMA,
            pltpu.SemaphoreType.DMA,
        ],
        compiler_params=pltpu.CompilerParams(use_tc_tiling_on_sc=False),
    )


def kernel(x, tables):
    b, nf = x.shape
    _, vocab, d = tables.shape
    total = b * nf
    x_flat = x.astype(jnp.int32).reshape(total // CHUNK, CHUNK)
    tab = tables.reshape(nf * vocab, d)
    out = _make_kernel(total)(x_flat, tab)
    return out.reshape(b, nf, d)
